# Initial kernel scaffold; baseline (speedup 1.0000x reference)
import jax
import jax.numpy as jnp
from jax.experimental import pallas as pl


def kernel(x, emb, W, b):
    raise NotImplementedError("write your pallas kernel here")


# speedup vs baseline: 5.4612x; 5.4612x over previous
import functools

import jax
import jax.numpy as jnp
from jax import lax
from jax.experimental import pallas as pl
from jax.experimental.pallas import tpu as pltpu
from jax.experimental.pallas import tpu_sc as plsc

_INFO = plsc.get_sparse_core_info()
_NC = _INFO.num_cores
_NS = _INFO.num_subcores
_NW = _NC * _NS


@functools.lru_cache(maxsize=None)
def _make_kernel(n: int):
    assert n % (8 * _NW) == 0, n
    per_worker = n // _NW
    chunk = per_worker
    for cand in (10240, 8192, 6400, 5120, 2560, 1280, 640, 320, 160, 80, 40, 16, 8):
        if per_worker % cand == 0 and cand * 12 <= 400 * 1024:
            chunk = cand
            break
    nchunks = per_worker // chunk
    mesh = plsc.VectorSubcoreMesh(core_axis_name="c", subcore_axis_name="s")

    @functools.partial(
        pl.kernel,
        mesh=mesh,
        compiler_params=pltpu.CompilerParams(needs_layout_passes=False),
        out_type=jax.ShapeDtypeStruct((2 * n,), jnp.float32),
        scratch_types=[
            pltpu.VMEM((16,), jnp.float32),
            pltpu.VMEM((16,), jnp.float32),
            pltpu.VMEM((16,), jnp.float32),
            pltpu.VMEM((chunk,), jnp.int32),
            pltpu.VMEM((2 * chunk,), jnp.float32)
        ],
    )
    def sc_kernel(x_hbm, params_hbm, out_hbm, params_v, tab0_v, tab1_v, x_v, out_v):
        wid = lax.axis_index("s") * _NC + lax.axis_index("c")
        lane = lax.iota(jnp.int32, 16)

        pltpu.sync_copy(params_hbm, params_v)
        v4 = lane & 3
        e0 = plsc.load_gather(params_v, [v4 * 2])
        e1 = plsc.load_gather(params_v, [v4 * 2 + 1])

        def splat(i):
            return plsc.load_gather(params_v, [jnp.full((16,), i, jnp.int32)])

        tab0_v[...] = e0 * splat(8) + e1 * splat(9) + splat(12)
        tab1_v[...] = e0 * splat(10) + e1 * splat(11) + splat(13)

        lane2 = lane * 2
        base = wid * per_worker
        for g in range(nchunks):
            off = base + g * chunk
            pltpu.sync_copy(x_hbm.at[pl.ds(off, chunk)], x_v)

            def body(i, carry):
                idx = x_v[pl.ds(i * 16, 16)]
                v0 = plsc.load_gather(tab0_v, [idx])
                v1 = plsc.load_gather(tab1_v, [idx])
                pos = lane2 + i * 32
                plsc.store_scatter(out_v, [pos], v0)
                plsc.store_scatter(out_v, [pos + 1], v1)
                return carry

            lax.fori_loop(0, chunk // 16, body, 0)
            pltpu.sync_copy(out_v, out_hbm.at[pl.ds(2 * off, 2 * chunk)])

    return sc_kernel


def kernel(x, emb, W, b):
    orig_shape = x.shape
    xf = x.reshape(-1).astype(jnp.int32)
    params = jnp.concatenate([
        emb.reshape(-1).astype(jnp.float32),
        W.reshape(-1).astype(jnp.float32),
        b.astype(jnp.float32),
        jnp.zeros((2,), jnp.float32),
    ])
    out = _make_kernel(xf.shape[0])(xf, params)
    return out.reshape(orig_shape + (2,))


# speedup vs baseline: 5.5598x; 1.0181x over previous
import functools

import jax
import jax.numpy as jnp
from jax import lax
from jax.experimental import pallas as pl
from jax.experimental.pallas import tpu as pltpu
from jax.experimental.pallas import tpu_sc as plsc

_INFO = plsc.get_sparse_core_info()
_NC = _INFO.num_cores
_NS = _INFO.num_subcores
_NW = _NC * _NS


@functools.lru_cache(maxsize=None)
def _make_kernel(n: int):
    assert n % (8 * _NW) == 0, n
    per_worker = n // _NW
    chunk = per_worker
    for cand in (10240, 8192, 6400, 5120, 2560, 1280, 640, 320, 160, 80, 40, 16, 8):
        if per_worker % cand == 0 and cand * 12 <= 400 * 1024:
            chunk = cand
            break
    nchunks = per_worker // chunk
    mesh = plsc.VectorSubcoreMesh(core_axis_name="c", subcore_axis_name="s")

    @functools.partial(
        pl.kernel,
        mesh=mesh,
        compiler_params=pltpu.CompilerParams(needs_layout_passes=False),
        out_type=jax.ShapeDtypeStruct((2 * n,), jnp.float32),
        scratch_types=[
            pltpu.VMEM((16,), jnp.float32),
            pltpu.VMEM((16,), jnp.float32),
            pltpu.VMEM((16,), jnp.float32),
            pltpu.VMEM((chunk,), jnp.int32),
            pltpu.VMEM((2 * chunk,), jnp.float32)
        ],
    )
    def sc_kernel(x_hbm, params_hbm, out_hbm, params_v, tab0_v, tab1_v, x_v, out_v):
        wid = lax.axis_index("s") * _NC + lax.axis_index("c")
        lane = lax.iota(jnp.int32, 16)

        pltpu.sync_copy(params_hbm, params_v)
        v4 = lane & 3
        e0 = plsc.load_gather(params_v, [v4 * 2])
        e1 = plsc.load_gather(params_v, [v4 * 2 + 1])

        def splat(i):
            return plsc.load_gather(params_v, [jnp.full((16,), i, jnp.int32)])

        tab0_v[...] = e0 * splat(8) + e1 * splat(9) + splat(12)
        tab1_v[...] = e0 * splat(10) + e1 * splat(11) + splat(13)

        lane2 = lane * 2
        base = wid * per_worker
        for g in range(nchunks):
            off = base + g * chunk
            pltpu.sync_copy(x_hbm.at[pl.ds(off, chunk)], x_v)

            @plsc.parallel_loop(0, chunk // 16, unroll=8)
            def body(i):
                idx = x_v[pl.ds(i * 16, 16)]
                v0 = plsc.load_gather(tab0_v, [idx])
                v1 = plsc.load_gather(tab1_v, [idx])
                pos = lane2 + i * 32
                plsc.store_scatter(out_v, [pos], v0)
                plsc.store_scatter(out_v, [pos + 1], v1)
            pltpu.sync_copy(out_v, out_hbm.at[pl.ds(2 * off, 2 * chunk)])

    return sc_kernel


def kernel(x, emb, W, b):
    orig_shape = x.shape
    xf = x.reshape(-1).astype(jnp.int32)
    params = jnp.concatenate([
        emb.reshape(-1).astype(jnp.float32),
        W.reshape(-1).astype(jnp.float32),
        b.astype(jnp.float32),
        jnp.zeros((2,), jnp.float32),
    ])
    out = _make_kernel(xf.shape[0])(xf, params)
    return out.reshape(orig_shape + (2,))


# speedup vs baseline: 179.1322x; 32.2190x over previous
import functools

import jax
import jax.numpy as jnp
from jax import lax
from jax.experimental import pallas as pl
from jax.experimental.pallas import tpu as pltpu
from jax.experimental.pallas import tpu_sc as plsc

_INFO = plsc.get_sparse_core_info()
_NC = _INFO.num_cores
_NS = _INFO.num_subcores
_NW = _NC * _NS


@functools.lru_cache(maxsize=None)
def _make_kernel(jh: int, ih: int):
    assert ih % _NW == 0, ih
    ih_per_worker = ih // _NW
    jh_chunk = 5
    assert jh % jh_chunk == 0
    njc = jh // jh_chunk
    rows = 8 * jh_chunk
    mesh = plsc.VectorSubcoreMesh(core_axis_name="c", subcore_axis_name="s")

    @functools.partial(
        pl.kernel,
        mesh=mesh,
        compiler_params=pltpu.CompilerParams(
            needs_layout_passes=False, use_tc_tiling_on_sc=False
        ),
        out_type=jax.ShapeDtypeStruct((8 * jh, ih, 2, 128), jnp.float32),
        scratch_types=[
            pltpu.VMEM((16,), jnp.float32),
            pltpu.VMEM((16,), jnp.float32),
            pltpu.VMEM((16,), jnp.float32),
            pltpu.VMEM((jh_chunk, 8, 128), jnp.int32),
            pltpu.VMEM((rows, 2, 128), jnp.float32),
        ],
    )
    def sc_kernel(x_hbm, params_hbm, out_hbm, params_v, tab0_v, tab1_v, x_v, out_v):
        wid = lax.axis_index("s") * _NC + lax.axis_index("c")
        lane = lax.iota(jnp.int32, 16)

        pltpu.sync_copy(params_hbm, params_v)
        v4 = lane & 3
        e0 = plsc.load_gather(params_v, [v4 * 2])
        e1 = plsc.load_gather(params_v, [v4 * 2 + 1])

        def splat(i):
            return plsc.load_gather(params_v, [jnp.full((16,), i, jnp.int32)])

        tab0_v[...] = e0 * splat(8) + e1 * splat(9) + splat(12)
        tab1_v[...] = e0 * splat(10) + e1 * splat(11) + splat(13)

        for u in range(ih_per_worker):
            ihi = wid + u * _NW
            for g in range(njc):
                jh0 = g * jh_chunk
                pltpu.sync_copy(x_hbm.at[pl.ds(jh0, jh_chunk), ihi], x_v)

                @plsc.parallel_loop(0, rows, unroll=2)
                def body(r):
                    jhl = r >> 3
                    jlo = r & 7
                    for l in range(8):
                        idx = x_v[jhl, jlo, pl.ds(l * 16, 16)]
                        v0 = plsc.load_gather(tab0_v, [idx])
                        v1 = plsc.load_gather(tab1_v, [idx])
                        out_v[r, 0, pl.ds(l * 16, 16)] = v0
                        out_v[r, 1, pl.ds(l * 16, 16)] = v1

                pltpu.sync_copy(out_v, out_hbm.at[pl.ds(8 * jh0, rows), ihi])

    return sc_kernel


def kernel(x, emb, W, b):
    nrows, ncols = x.shape
    jh, ih = ncols // 8, nrows // 128
    params = jnp.concatenate([
        emb.reshape(-1).astype(jnp.float32),
        W.reshape(-1).astype(jnp.float32),
        b.astype(jnp.float32),
        jnp.zeros((2,), jnp.float32),
    ])
    x4 = x.astype(jnp.int32).T.reshape(jh, 8, ih, 128).transpose(0, 2, 1, 3)
    out4 = _make_kernel(jh, ih)(x4, params)
    return out4.transpose(1, 3, 0, 2).reshape(nrows, ncols, 2)


# speedup vs baseline: 253.4942x; 1.4151x over previous
import functools

import jax
import jax.numpy as jnp
from jax import lax
from jax.experimental import pallas as pl
from jax.experimental.pallas import tpu as pltpu
from jax.experimental.pallas import tpu_sc as plsc

_INFO = plsc.get_sparse_core_info()
_NC = _INFO.num_cores
_NS = _INFO.num_subcores
_NW = _NC * _NS


@functools.lru_cache(maxsize=None)
def _make_kernel(jh: int, ih: int):
    assert ih % _NW == 0, ih
    ih_per_worker = ih // _NW
    jh_chunk = 5
    assert jh % jh_chunk == 0
    njc = jh // jh_chunk
    rows = 8 * jh_chunk
    mesh = plsc.VectorSubcoreMesh(core_axis_name="c", subcore_axis_name="s")

    @functools.partial(
        pl.kernel,
        mesh=mesh,
        compiler_params=pltpu.CompilerParams(
            needs_layout_passes=False, use_tc_tiling_on_sc=False
        ),
        out_type=jax.ShapeDtypeStruct((8 * jh, ih, 2, 128), jnp.float32),
        scratch_types=[
            pltpu.VMEM((16,), jnp.float32),
            pltpu.VMEM((16,), jnp.float32),
            pltpu.VMEM((16,), jnp.float32),
            pltpu.VMEM((jh_chunk, 8, 128), jnp.int32),
            pltpu.VMEM((jh_chunk, 8, 128), jnp.int32),
            pltpu.VMEM((rows, 2, 128), jnp.float32),
            pltpu.VMEM((rows, 2, 128), jnp.float32),
            pltpu.SemaphoreType.DMA,
            pltpu.SemaphoreType.DMA,
            pltpu.SemaphoreType.DMA,
            pltpu.SemaphoreType.DMA,
        ],
    )
    def sc_kernel(x_hbm, params_hbm, out_hbm, params_v, tab0_v, tab1_v,
                  x_v0, x_v1, out_v0, out_v1, isem0, isem1, osem0, osem1):
        wid = lax.axis_index("s") * _NC + lax.axis_index("c")
        lane = lax.iota(jnp.int32, 16)

        pltpu.sync_copy(params_hbm, params_v)
        v4 = lane & 3
        e0 = plsc.load_gather(params_v, [v4 * 2])
        e1 = plsc.load_gather(params_v, [v4 * 2 + 1])

        def splat(i):
            return plsc.load_gather(params_v, [jnp.full((16,), i, jnp.int32)])

        tab0_v[...] = e0 * splat(8) + e1 * splat(9) + splat(12)
        tab1_v[...] = e0 * splat(10) + e1 * splat(11) + splat(13)

        x_bufs = (x_v0, x_v1)
        out_bufs = (out_v0, out_v1)
        isems = (isem0, isem1)
        osems = (osem0, osem1)
        chunks = [
            (wid + u * _NW, g * jh_chunk)
            for u in range(ih_per_worker)
            for g in range(njc)
        ]
        nt = len(chunks)

        def start_in(t):
            ihi, jh0 = chunks[t]
            b = t & 1
            return pltpu.async_copy(
                x_hbm.at[pl.ds(jh0, jh_chunk), ihi], x_bufs[b], isems[b]
            )

        in_cps = {0: start_in(0)}
        out_cps = {}
        for t in range(nt):
            b = t & 1
            if t + 1 < nt:
                in_cps[t + 1] = start_in(t + 1)
            in_cps.pop(t).wait()
            if t >= 2:
                out_cps.pop(t - 2).wait()
            x_v = x_bufs[b]
            out_v = out_bufs[b]

            @plsc.parallel_loop(0, rows, unroll=2)
            def body(r):
                jhl = r >> 3
                jlo = r & 7
                for l in range(8):
                    idx = x_v[jhl, jlo, pl.ds(l * 16, 16)]
                    v0 = plsc.load_gather(tab0_v, [idx])
                    v1 = plsc.load_gather(tab1_v, [idx])
                    out_v[r, 0, pl.ds(l * 16, 16)] = v0
                    out_v[r, 1, pl.ds(l * 16, 16)] = v1

            ihi, jh0 = chunks[t]
            out_cps[t] = pltpu.async_copy(
                out_v, out_hbm.at[pl.ds(8 * jh0, rows), ihi], osems[b]
            )
        for t in sorted(out_cps):
            out_cps.pop(t).wait()

    return sc_kernel


def kernel(x, emb, W, b):
    nrows, ncols = x.shape
    jh, ih = ncols // 8, nrows // 128
    params = jnp.concatenate([
        emb.reshape(-1).astype(jnp.float32),
        W.reshape(-1).astype(jnp.float32),
        b.astype(jnp.float32),
        jnp.zeros((2,), jnp.float32),
    ])
    x4 = x.astype(jnp.int32).T.reshape(jh, 8, ih, 128).transpose(0, 2, 1, 3)
    out4 = _make_kernel(jh, ih)(x4, params)
    return out4.transpose(1, 3, 0, 2).reshape(nrows, ncols, 2)
